# baseline (speedup 1.0000x reference)
import functools

import jax
import jax.numpy as jnp
from jax import lax
from jax.experimental import pallas as pl
from jax.experimental.pallas import tpu as pltpu
from jax.experimental.pallas import tpu_sc as plsc

_NC = 2
_NS = 16
_NW = _NC * _NS


def _tc_part(W_pos, nbatch, seq, d):
    blk = 512

    def body(w_ref, o_ref):
        o_ref[...] = jnp.broadcast_to(w_ref[...][None], o_ref.shape)

    return pl.pallas_call(
        body,
        grid=(seq // blk,),
        in_specs=[pl.BlockSpec((blk, d), lambda j: (j, 0))],
        out_specs=pl.BlockSpec((nbatch, blk, d), lambda j: (0, j, 0)),
        out_shape=jax.ShapeDtypeStruct((nbatch, seq, d), W_pos.dtype),
    )(W_pos)


def _make_sc_part(nbatch, seq, d, dtype):
    rows_per_w = seq // _NW
    ch = 32
    nch = rows_per_w // ch
    mesh = plsc.VectorSubcoreMesh(core_axis_name="c", subcore_axis_name="s")

    @functools.partial(
        pl.kernel,
        mesh=mesh,
        out_type=jax.ShapeDtypeStruct((nbatch, seq, d), dtype),
        scratch_types=[
            pltpu.VMEM((ch, d), dtype),
            pltpu.VMEM((ch, d), dtype),
            pltpu.SemaphoreType.DMA,
            pltpu.SemaphoreType.DMA,
        ],
    )
    def k(w_hbm, out_hbm, buf0, buf1, rsem, wsem):
        wid = lax.axis_index("s") * _NC + lax.axis_index("c")
        base = wid * rows_per_w
        bufs = (buf0, buf1)

        rd = pltpu.async_copy(w_hbm.at[pl.ds(base, ch), :], buf0, rsem)
        rd.wait()
        for c in range(nch):
            cur = bufs[c % 2]
            nxt = bufs[(c + 1) % 2]
            if c + 1 < nch:
                rd_next = pltpu.async_copy(
                    w_hbm.at[pl.ds(base + (c + 1) * ch, ch), :], nxt, rsem
                )
            wrs = [
                pltpu.async_copy(
                    cur, out_hbm.at[b, pl.ds(base + c * ch, ch), :], wsem
                )
                for b in range(nbatch)
            ]
            for w in wrs:
                w.wait()
            if c + 1 < nch:
                rd_next.wait()

    return k


def kernel(tokens, W_pos):
    batch, seq = tokens.shape
    d = W_pos.shape[1]
    tc_batches = batch // 2
    sc_batches = batch - tc_batches
    tc_out = _tc_part(W_pos, tc_batches, seq, d)
    sc_out = _make_sc_part(sc_batches, seq, d, W_pos.dtype)(W_pos)
    return jnp.concatenate([tc_out, sc_out], axis=0)


# speedup vs baseline: 3.5114x; 3.5114x over previous
import jax
import jax.numpy as jnp
from jax.experimental import pallas as pl


def _copy_body(w_ref, o_ref):
    o_ref[...] = jnp.broadcast_to(w_ref[...][None], o_ref.shape)


def kernel(tokens, W_pos):
    batch, seq = tokens.shape
    d = W_pos.shape[1]
    blk = 1024
    out = pl.pallas_call(
        _copy_body,
        grid=(seq // blk,),
        in_specs=[pl.BlockSpec((blk, d), lambda j: (j, 0))],
        out_specs=pl.BlockSpec((batch, blk, d), lambda j: (0, j, 0)),
        out_shape=jax.ShapeDtypeStruct((batch, seq, d), W_pos.dtype),
    )(W_pos)
    return out
